# baseline (speedup 1.0000x reference)
import functools

import jax
import jax.numpy as jnp
from jax import lax
from jax.experimental import pallas as pl
from jax.experimental.pallas import tpu as pltpu
from jax.experimental.pallas import tpu_sc as plsc

_RB = 16
_NAGG = 8
_ND = 4


def _norm16(b):
    n = jnp.sqrt(jnp.sum(b * b, axis=1, keepdims=True))
    inv = 1.0 / jnp.maximum(n, 1e-12)
    return (b * inv).astype(jnp.bfloat16)


def _main_body(nxb, x_hbm, w_hbm, age_ref, idx_ref,
               buf_ref, sems, xh_ref, wnh_ref, gmax_ref, gidx_ref):
    s = pl.program_id(0)
    ns = pl.num_programs(0)
    grp = _RB * _NAGG

    def _issue(b):
        slot = lax.rem(b, _ND)

        @pl.when(b < nxb)
        def _from_x():
            pltpu.make_async_copy(
                x_hbm.at[b], buf_ref.at[slot], sems.at[slot]).start()

        @pl.when((b >= nxb) & (b < ns))
        def _from_w():
            pltpu.make_async_copy(
                w_hbm.at[b - nxb], buf_ref.at[slot], sems.at[slot]).start()

    @pl.when(s == 0)
    def _prologue():
        for b in range(_ND - 1):
            _issue(jnp.int32(b))

    slot = lax.rem(s, _ND)
    pltpu.make_async_copy(
        x_hbm.at[0], buf_ref.at[slot], sems.at[slot]).wait()

    _issue(s + _ND - 1)

    @pl.when(s < nxb)
    def _x_phase():
        xh_ref[pl.ds(s * _RB, _RB), :] = _norm16(buf_ref[slot])

    @pl.when(s >= nxb)
    def _w_phase():
        sw = s - nxb
        i = lax.rem(sw, _NAGG)
        j = lax.div(sw, _NAGG)
        wnh_ref[pl.ds(i * _RB, _RB), :] = _norm16(buf_ref[slot])

        @pl.when(i == _NAGG - 1)
        def _dot():
            sc = lax.dot_general(
                xh_ref[...], wnh_ref[...], (((1,), (1,)), ((), ())),
                preferred_element_type=jnp.float32)
            act = (age_ref[0] >= 1.0).astype(jnp.float32)
            sc = sc * act
            bm = jnp.max(sc, axis=1, keepdims=True)
            ii = lax.broadcasted_iota(jnp.int32, sc.shape, 1) + j * grp
            li = jnp.min(jnp.where(sc == bm, ii, jnp.int32(2**30)),
                         axis=1, keepdims=True)

            @pl.when(j == 0)
            def _first():
                gmax_ref[...] = bm
                gidx_ref[...] = li

            @pl.when(j > 0)
            def _update():
                better = bm > gmax_ref[...]
                gidx_ref[...] = jnp.where(better, li, gidx_ref[...])
                gmax_ref[...] = jnp.maximum(bm, gmax_ref[...])

            @pl.when(s == ns - 1)
            def _emit():
                idx_ref[...] = gidx_ref[...]


def _scores_argmax(xf, W, age_row):
    B, K = xf.shape
    Y = W.shape[0]
    grp = _RB * _NAGG
    nxb = B // _RB
    nwb = Y // _RB
    X3 = xf.reshape(nxb, _RB, K)
    W3 = W.reshape(nwb, _RB, K)
    return pl.pallas_call(
        functools.partial(_main_body, nxb),
        grid=(nxb + nwb,),
        in_specs=[
            pl.BlockSpec(memory_space=pl.ANY),
            pl.BlockSpec(memory_space=pl.ANY),
            pl.BlockSpec((1, 1, grp),
                         lambda s: (jnp.maximum(s - nxb, 0) // _NAGG, 0, 0)),
        ],
        out_specs=pl.BlockSpec((B, 1), lambda s: (0, 0)),
        out_shape=jax.ShapeDtypeStruct((B, 1), jnp.int32),
        scratch_shapes=[
            pltpu.VMEM((_ND, _RB, K), jnp.float32),
            pltpu.SemaphoreType.DMA((_ND,)),
            pltpu.VMEM((B, K), jnp.bfloat16),
            pltpu.VMEM((grp, K), jnp.bfloat16),
            pltpu.VMEM((B, 1), jnp.float32),
            pltpu.VMEM((B, 1), jnp.int32),
        ],
    )(X3, W3, age_row.reshape(Y // grp, 1, grp))


def _sc_gather(table, idx):
    Yp, D = table.shape
    B = idx.shape[0]
    info = plsc.get_sparse_core_info()
    nw = info.num_cores * info.num_subcores
    bpw = B // nw
    mesh = plsc.VectorSubcoreMesh(core_axis_name="c", subcore_axis_name="s")

    @functools.partial(
        pl.kernel, mesh=mesh,
        out_type=jax.ShapeDtypeStruct((B, D), jnp.float32),
        scratch_types=[
            pltpu.VMEM((bpw,), jnp.int32),
            pltpu.VMEM((bpw, D), jnp.float32),
            pltpu.SemaphoreType.DMA,
        ],
    )
    def gk(table_hbm, idx_hbm, out_hbm, idx_v, rows_v, sem):
        wid = lax.axis_index("s") * info.num_cores + lax.axis_index("c")
        base = wid * bpw
        pltpu.sync_copy(idx_hbm.at[pl.ds(base, bpw)], idx_v)
        pltpu.async_copy(table_hbm.at[idx_v], rows_v, sem).wait()
        pltpu.sync_copy(rows_v, out_hbm.at[pl.ds(base, bpw)])

    return gk(table, idx)


def kernel(x, z, per_item, epo, x2, x3, x4, W_x2y, W_y2z, W_x2y4, y_neuron_age):
    B = x.shape[0]
    xf = x.reshape(B, -1)
    idx = _scores_argmax(xf, W_x2y, y_neuron_age)[:, 0]
    Z, Y = W_y2z.shape
    Dp = ((Z + 127) // 128) * 128
    table = jnp.zeros((Y, Dp), jnp.float32).at[:, :Z].set(W_y2z.T)
    out = _sc_gather(table, idx)
    return out[:, :Z]


# speedup vs baseline: 1.1515x; 1.1515x over previous
import functools

import jax
import jax.numpy as jnp
from jax import lax
from jax.experimental import pallas as pl
from jax.experimental.pallas import tpu as pltpu
from jax.experimental.pallas import tpu_sc as plsc

_RBX = 32
_RBW = 16
_NAGG = 8
_ND = 6


def _xnorm_body(x_ref, xh_ref):
    xb = x_ref[...]
    n = jnp.sqrt(jnp.sum(xb * xb, axis=1, keepdims=True))
    inv = 1.0 / jnp.maximum(n, 1e-12)
    xh_ref[...] = (xb * inv).astype(jnp.bfloat16)


def _xnorm(xf):
    B, K = xf.shape
    return pl.pallas_call(
        _xnorm_body,
        grid=(B // _RBX,),
        in_specs=[pl.BlockSpec((_RBX, K), lambda k: (k, 0))],
        out_specs=pl.BlockSpec((_RBX, K), lambda k: (k, 0)),
        out_shape=jax.ShapeDtypeStruct((B, K), jnp.bfloat16),
    )(xf)


def _wmain_body(w_hbm, xh_ref, age_ref, idx_ref,
                buf_ref, sems, wnh_ref, gmax_ref, gidx_ref):
    s = pl.program_id(0)
    ns = pl.num_programs(0)
    grp = _RBW * _NAGG
    i = lax.rem(s, _NAGG)
    j = lax.div(s, _NAGG)

    def _issue(b):
        slot = lax.rem(b, _ND)
        pltpu.make_async_copy(
            w_hbm.at[b], buf_ref.at[slot], sems.at[slot]).start()

    @pl.when(s == 0)
    def _prologue():
        for b in range(_ND - 1):
            _issue(jnp.int32(b))

    slot = lax.rem(s, _ND)
    pltpu.make_async_copy(
        w_hbm.at[s], buf_ref.at[slot], sems.at[slot]).wait()

    wb = buf_ref[slot]
    n = jnp.sqrt(jnp.sum(wb * wb, axis=1, keepdims=True))
    inv = 1.0 / jnp.maximum(n, 1e-12)
    wnh_ref[pl.ds(i * _RBW, _RBW), :] = (wb * inv).astype(jnp.bfloat16)

    @pl.when(s + _ND - 1 < ns)
    def _refill():
        _issue(s + _ND - 1)

    @pl.when(i == _NAGG - 1)
    def _dot():
        sc = lax.dot_general(
            xh_ref[...], wnh_ref[...], (((1,), (1,)), ((), ())),
            preferred_element_type=jnp.float32)
        act = (age_ref[0] >= 1.0).astype(jnp.float32)
        sc = sc * act
        bm = jnp.max(sc, axis=1, keepdims=True)
        ii = lax.broadcasted_iota(jnp.int32, sc.shape, 1) + j * grp
        li = jnp.min(jnp.where(sc == bm, ii, jnp.int32(2**30)),
                     axis=1, keepdims=True)

        @pl.when(j == 0)
        def _first():
            gmax_ref[...] = bm
            gidx_ref[...] = li

        @pl.when(j > 0)
        def _update():
            better = bm > gmax_ref[...]
            gidx_ref[...] = jnp.where(better, li, gidx_ref[...])
            gmax_ref[...] = jnp.maximum(bm, gmax_ref[...])

        @pl.when(s == ns - 1)
        def _emit():
            idx_ref[...] = gidx_ref[...]


def _scores_argmax(xf, W, age_row):
    B, K = xf.shape
    Y = W.shape[0]
    grp = _RBW * _NAGG
    nsteps = Y // _RBW
    W3 = W.reshape(nsteps, _RBW, K)
    xh = _xnorm(xf)
    return pl.pallas_call(
        _wmain_body,
        grid=(nsteps,),
        in_specs=[
            pl.BlockSpec(memory_space=pl.ANY),
            pl.BlockSpec((B, K), lambda s: (0, 0)),
            pl.BlockSpec((1, 1, grp), lambda s: (s // _NAGG, 0, 0)),
        ],
        out_specs=pl.BlockSpec((B, 1), lambda s: (0, 0)),
        out_shape=jax.ShapeDtypeStruct((B, 1), jnp.int32),
        scratch_shapes=[
            pltpu.VMEM((_ND, _RBW, K), jnp.float32),
            pltpu.SemaphoreType.DMA((_ND,)),
            pltpu.VMEM((grp, K), jnp.bfloat16),
            pltpu.VMEM((B, 1), jnp.float32),
            pltpu.VMEM((B, 1), jnp.int32),
        ],
    )(W3, xh, age_row.reshape(Y // grp, 1, grp))


def _sc_gather(table, idx):
    Yp, D = table.shape
    B = idx.shape[0]
    info = plsc.get_sparse_core_info()
    nw = info.num_cores * info.num_subcores
    bpw = B // nw
    mesh = plsc.VectorSubcoreMesh(core_axis_name="c", subcore_axis_name="s")

    @functools.partial(
        pl.kernel, mesh=mesh,
        out_type=jax.ShapeDtypeStruct((B, D), jnp.float32),
        scratch_types=[
            pltpu.VMEM((bpw,), jnp.int32),
            pltpu.VMEM((bpw, D), jnp.float32),
            pltpu.SemaphoreType.DMA,
        ],
    )
    def gk(table_hbm, idx_hbm, out_hbm, idx_v, rows_v, sem):
        wid = lax.axis_index("s") * info.num_cores + lax.axis_index("c")
        base = wid * bpw
        pltpu.sync_copy(idx_hbm.at[pl.ds(base, bpw)], idx_v)
        pltpu.async_copy(table_hbm.at[idx_v], rows_v, sem).wait()
        pltpu.sync_copy(rows_v, out_hbm.at[pl.ds(base, bpw)])

    return gk(table, idx)


def kernel(x, z, per_item, epo, x2, x3, x4, W_x2y, W_y2z, W_x2y4, y_neuron_age):
    B = x.shape[0]
    xf = x.reshape(B, -1)
    idx = _scores_argmax(xf, W_x2y, y_neuron_age)[:, 0]
    Z, Y = W_y2z.shape
    Dp = ((Z + 127) // 128) * 128
    table = jnp.zeros((Y, Dp), jnp.float32).at[:, :Z].set(W_y2z.T)
    out = _sc_gather(table, idx)
    return out[:, :Z]


# speedup vs baseline: 1.1547x; 1.0028x over previous
import functools

import jax
import jax.numpy as jnp
from jax import lax
from jax.experimental import pallas as pl
from jax.experimental.pallas import tpu as pltpu
from jax.experimental.pallas import tpu_sc as plsc

_RBX = 32
_RBW = 16
_NAGG = 8
_ND = 6


def _xnorm_body(x_ref, xh_ref):
    xb = x_ref[...]
    n = jnp.sqrt(jnp.sum(xb * xb, axis=1, keepdims=True))
    inv = 1.0 / jnp.maximum(n, 1e-12)
    xh_ref[...] = (xb * inv).astype(jnp.bfloat16)


def _xnorm(xf):
    B, K = xf.shape
    return pl.pallas_call(
        _xnorm_body,
        grid=(B // _RBX,),
        in_specs=[pl.BlockSpec((_RBX, K), lambda k: (k, 0))],
        out_specs=pl.BlockSpec((_RBX, K), lambda k: (k, 0)),
        out_shape=jax.ShapeDtypeStruct((B, K), jnp.bfloat16),
    )(xf)


def _wmain_body(w_hbm, xh_hbm, age_ref, idx_ref,
                buf_ref, sems, xh_ref, xsem, wnh_ref, gmax_ref, gidx_ref):
    s = pl.program_id(0)
    ns = pl.num_programs(0)
    grp = _RBW * _NAGG
    i = lax.rem(s, _NAGG)
    j = lax.div(s, _NAGG)

    def _issue(b):
        slot = lax.rem(b, _ND)
        pltpu.make_async_copy(
            w_hbm.at[b], buf_ref.at[slot], sems.at[slot]).start()

    @pl.when(s == 0)
    def _prologue():
        pltpu.make_async_copy(xh_hbm, xh_ref, xsem).start()
        for b in range(_ND - 1):
            _issue(jnp.int32(b))

    slot = lax.rem(s, _ND)
    pltpu.make_async_copy(
        w_hbm.at[s], buf_ref.at[slot], sems.at[slot]).wait()

    wb = buf_ref[slot]
    n = jnp.sqrt(jnp.sum(wb * wb, axis=1, keepdims=True))
    inv = 1.0 / jnp.maximum(n, 1e-12)
    wnh_ref[pl.ds(i * _RBW, _RBW), :] = (wb * inv).astype(jnp.bfloat16)

    @pl.when(s + _ND - 1 < ns)
    def _refill():
        _issue(s + _ND - 1)

    @pl.when(i == _NAGG - 1)
    def _dot():
        @pl.when(j == 0)
        def _xwait():
            pltpu.make_async_copy(xh_hbm, xh_ref, xsem).wait()

        sc = lax.dot_general(
            xh_ref[...], wnh_ref[...], (((1,), (1,)), ((), ())),
            preferred_element_type=jnp.float32)
        act = (age_ref[0] >= 1.0).astype(jnp.float32)
        sc = sc * act
        bm = jnp.max(sc, axis=1, keepdims=True)
        ii = lax.broadcasted_iota(jnp.int32, sc.shape, 1) + j * grp
        li = jnp.min(jnp.where(sc == bm, ii, jnp.int32(2**30)),
                     axis=1, keepdims=True)

        @pl.when(j == 0)
        def _first():
            gmax_ref[...] = bm
            gidx_ref[...] = li

        @pl.when(j > 0)
        def _update():
            better = bm > gmax_ref[...]
            gidx_ref[...] = jnp.where(better, li, gidx_ref[...])
            gmax_ref[...] = jnp.maximum(bm, gmax_ref[...])

        @pl.when(s == ns - 1)
        def _emit():
            idx_ref[...] = gidx_ref[...]


def _scores_argmax(xf, W, age_row):
    B, K = xf.shape
    Y = W.shape[0]
    grp = _RBW * _NAGG
    nsteps = Y // _RBW
    W3 = W.reshape(nsteps, _RBW, K)
    xh = _xnorm(xf)
    return pl.pallas_call(
        _wmain_body,
        grid=(nsteps,),
        in_specs=[
            pl.BlockSpec(memory_space=pl.ANY),
            pl.BlockSpec(memory_space=pl.ANY),
            pl.BlockSpec((1, 1, grp), lambda s: (s // _NAGG, 0, 0)),
        ],
        out_specs=pl.BlockSpec((B, 1), lambda s: (0, 0)),
        out_shape=jax.ShapeDtypeStruct((B, 1), jnp.int32),
        scratch_shapes=[
            pltpu.VMEM((_ND, _RBW, K), jnp.float32),
            pltpu.SemaphoreType.DMA((_ND,)),
            pltpu.VMEM((B, K), jnp.bfloat16),
            pltpu.SemaphoreType.DMA,
            pltpu.VMEM((grp, K), jnp.bfloat16),
            pltpu.VMEM((B, 1), jnp.float32),
            pltpu.VMEM((B, 1), jnp.int32),
        ],
    )(W3, xh, age_row.reshape(Y // grp, 1, grp))


def _sc_gather(table, idx):
    Yp, D = table.shape
    B = idx.shape[0]
    info = plsc.get_sparse_core_info()
    nw = info.num_cores * info.num_subcores
    bpw = B // nw
    mesh = plsc.VectorSubcoreMesh(core_axis_name="c", subcore_axis_name="s")

    @functools.partial(
        pl.kernel, mesh=mesh,
        out_type=jax.ShapeDtypeStruct((B, D), jnp.float32),
        scratch_types=[
            pltpu.VMEM((bpw,), jnp.int32),
            pltpu.VMEM((bpw, D), jnp.float32),
            pltpu.SemaphoreType.DMA,
        ],
    )
    def gk(table_hbm, idx_hbm, out_hbm, idx_v, rows_v, sem):
        wid = lax.axis_index("s") * info.num_cores + lax.axis_index("c")
        base = wid * bpw
        pltpu.sync_copy(idx_hbm.at[pl.ds(base, bpw)], idx_v)
        pltpu.async_copy(table_hbm.at[idx_v], rows_v, sem).wait()
        pltpu.sync_copy(rows_v, out_hbm.at[pl.ds(base, bpw)])

    return gk(table, idx)


def kernel(x, z, per_item, epo, x2, x3, x4, W_x2y, W_y2z, W_x2y4, y_neuron_age):
    B = x.shape[0]
    xf = x.reshape(B, -1)
    idx = _scores_argmax(xf, W_x2y, y_neuron_age)[:, 0]
    Z, Y = W_y2z.shape
    Dp = ((Z + 127) // 128) * 128
    table = jnp.zeros((Y, Dp), jnp.float32).at[:, :Z].set(W_y2z.T)
    out = _sc_gather(table, idx)
    return out[:, :Z]
